# baseline (speedup 1.0000x reference)
import jax
import jax.numpy as jnp
from jax import lax
from jax.experimental import pallas as pl
from jax.experimental.pallas import tpu as pltpu
from jax.experimental.pallas import tpu_sc as plsc

NUM_EMBEDDINGS = 1000000
EMBEDDING_DIM = 32
BATCH = 4096
SEQ_LEN = 200

_NW = 32
_B = BATCH * SEQ_LEN
_PER_W = _B // _NW
_CHUNK = 1600
_NCHUNK = _PER_W // _CHUNK

_VW = 512
_BLK_W = 61
_V_PER_W = _VW * _BLK_W
_TAIL_V0 = _NW * _V_PER_W
_TW = 128


def _transpose_body(tt_hbm, flat_hbm, t0, t1, o0, o1, si0, si1, so0, so1):
    cid = lax.axis_index("c")
    sid = lax.axis_index("s")
    wid = sid * 2 + cid
    v_base = wid * _V_PER_W

    tbuf = (t0, t1)
    obuf = (o0, o1)
    sin = (si0, si1)
    sout = (so0, so1)
    lanes = jax.lax.iota(jnp.int32, 16)
    hi = lanes + 16
    zeros = lanes * 0

    scat = lanes * 32

    def transpose_block(tb, ob, width):
        @plsc.parallel_loop(0, width // 16, 1, unroll=2)
        def _(g):
            base = g * 16
            off = scat + base * 32
            for c in range(32):
                x = tb[c, pl.ds(base, 16)]
                plsc.store_scatter(ob, [off + c], x)

    def start_in(blk, par):
        pltpu.async_copy(
            tt_hbm.at[:, pl.ds(v_base + blk * _VW, _VW)],
            tbuf[par].at[:, pl.ds(0, _VW)],
            sin[par],
        )

    start_in(0, 0)
    start_in(1, 1)

    def step(j, carry):
        for par in range(2):
            blk = 2 * j + par
            pltpu.make_async_copy(
                tt_hbm.at[:, pl.ds(v_base, _VW)],
                tbuf[par].at[:, pl.ds(0, _VW)],
                sin[par],
            ).wait()

            @pl.when(j > 0)
            def _():
                pltpu.make_async_copy(
                    obuf[par], flat_hbm.at[pl.ds(v_base * 32, _VW * 32)],
                    sout[par],
                ).wait()

            transpose_block(tbuf[par], obuf[par], _VW)
            pltpu.async_copy(
                obuf[par],
                flat_hbm.at[pl.ds((v_base + blk * _VW) * 32, _VW * 32)],
                sout[par],
            )

            @pl.when(blk + 2 < _BLK_W - 1)
            def _():
                pltpu.async_copy(
                    tt_hbm.at[:, pl.ds(v_base + (blk + 2) * _VW, _VW)],
                    tbuf[par].at[:, pl.ds(0, _VW)],
                    sin[par],
                )

        return carry

    lax.fori_loop(0, (_BLK_W - 1) // 2, step, 0)
    for par in range(2):
        pltpu.make_async_copy(
            obuf[par], flat_hbm.at[pl.ds(v_base * 32, _VW * 32)], sout[par]
        ).wait()

    v0p = v_base + (_BLK_W - 1) * _VW
    pltpu.sync_copy(
        tt_hbm.at[:, pl.ds(v0p, _VW)], tbuf[0].at[:, pl.ds(0, _VW)]
    )
    transpose_block(tbuf[0], obuf[0], _VW)
    pltpu.sync_copy(obuf[0], flat_hbm.at[pl.ds(v0p * 32, _VW * 32)])

    for k in range(4):

        @pl.when(wid == k)
        def _():
            v0 = _TAIL_V0 + _TW * k
            pltpu.sync_copy(
                tt_hbm.at[:, pl.ds(v0, _TW)], tbuf[0].at[:, pl.ds(0, _TW)]
            )
            transpose_block(tbuf[0], obuf[0], _TW)
            pltpu.sync_copy(
                obuf[0].at[pl.ds(0, _TW * 32)],
                flat_hbm.at[pl.ds(v0 * 32, _TW * 32)],
            )


def _gather_body(idx_hbm, table_hbm, out_hbm, idx_all, rows0, rows1,
                 g0, g1, w0, w1):
    cid = lax.axis_index("c")
    sid = lax.axis_index("s")
    wid = sid * 2 + cid
    w_base = wid * _PER_W

    pltpu.sync_copy(idx_hbm.at[wid], idx_all)

    rows = (rows0, rows1)
    gsem = (g0, g1)
    wsem = (w0, w1)
    gdesc = [None] * _NCHUNK
    wdesc = [None] * _NCHUNK
    gdesc[0] = pltpu.async_copy(table_hbm.at[idx_all.at[0]], rows[0], gsem[0])
    gdesc[1] = pltpu.async_copy(table_hbm.at[idx_all.at[1]], rows[1], gsem[1])
    for j in range(_NCHUNK):
        b = j % 2
        gdesc[j].wait()
        wdesc[j] = pltpu.async_copy(
            rows[b], out_hbm.at[pl.ds(w_base + j * _CHUNK, _CHUNK)], wsem[b]
        )
        if j + 2 < _NCHUNK:
            wdesc[j].wait()
            gdesc[j + 2] = pltpu.async_copy(
                table_hbm.at[idx_all.at[j + 2]], rows[b], gsem[b]
            )
    wdesc[_NCHUNK - 2].wait()
    wdesc[_NCHUNK - 1].wait()


@jax.jit
def kernel(token_ids, embedding):
    mesh = plsc.VectorSubcoreMesh(core_axis_name="c", subcore_axis_name="s")

    flat_table = pl.kernel(
        _transpose_body,
        out_type=jax.ShapeDtypeStruct((NUM_EMBEDDINGS * EMBEDDING_DIM,),
                                      jnp.float32),
        mesh=mesh,
        scratch_types=[
            pltpu.VMEM((EMBEDDING_DIM, _VW + 1), jnp.float32),
            pltpu.VMEM((EMBEDDING_DIM, _VW + 1), jnp.float32),
            pltpu.VMEM((_VW * EMBEDDING_DIM,), jnp.float32),
            pltpu.VMEM((_VW * EMBEDDING_DIM,), jnp.float32),
            pltpu.SemaphoreType.DMA,
            pltpu.SemaphoreType.DMA,
            pltpu.SemaphoreType.DMA,
            pltpu.SemaphoreType.DMA,
        ],
        compiler_params=pltpu.CompilerParams(
            use_tc_tiling_on_sc=True, needs_layout_passes=False,
            disable_bounds_checks=True
        ),
    )(embedding.T)

    tail_v0 = _TAIL_V0 + 4 * _TW
    tail_rows = jax.lax.slice(
        embedding, (tail_v0, 0), (NUM_EMBEDDINGS, EMBEDDING_DIM)
    ).reshape(-1)
    flat_table = jax.lax.dynamic_update_slice(
        flat_table, tail_rows, (tail_v0 * EMBEDDING_DIM,)
    )

    table_lin = flat_table.reshape(NUM_EMBEDDINGS, EMBEDDING_DIM)
    flat_ids = token_ids.reshape(_NW, _NCHUNK, _CHUNK).astype(jnp.int32)

    out = pl.kernel(
        _gather_body,
        out_type=jax.ShapeDtypeStruct((_B, EMBEDDING_DIM), jnp.float32),
        mesh=mesh,
        scratch_types=[
            pltpu.VMEM((_NCHUNK, _CHUNK), jnp.int32),
            pltpu.VMEM((_CHUNK, EMBEDDING_DIM), jnp.float32),
            pltpu.VMEM((_CHUNK, EMBEDDING_DIM), jnp.float32),
            pltpu.SemaphoreType.DMA,
            pltpu.SemaphoreType.DMA,
            pltpu.SemaphoreType.DMA,
            pltpu.SemaphoreType.DMA,
        ],
        compiler_params=pltpu.CompilerParams(
            use_tc_tiling_on_sc=False, disable_bounds_checks=True
        ),
    )(flat_ids, table_lin)
    return out.reshape(BATCH, SEQ_LEN, EMBEDDING_DIM)


# speedup vs baseline: 1.1185x; 1.1185x over previous
import jax
import jax.numpy as jnp
from jax import lax
from jax.experimental import pallas as pl
from jax.experimental.pallas import tpu as pltpu
from jax.experimental.pallas import tpu_sc as plsc

NUM_EMBEDDINGS = 1000000
EMBEDDING_DIM = 32
BATCH = 4096
SEQ_LEN = 200

_NW = 32
_B = BATCH * SEQ_LEN
_PER_W = _B // _NW
_CHUNK = 1600
_NCHUNK = _PER_W // _CHUNK

_VW = 512
_BLK_W = 61
_V_PER_W = _VW * _BLK_W
_TAIL_V0 = _NW * _V_PER_W
_TW = 128


def _transpose_body(tt_hbm, flat_hbm, t0, t1, o0, o1, si0, si1, so0, so1):
    cid = lax.axis_index("c")
    sid = lax.axis_index("s")
    wid = sid * 2 + cid
    v_base = wid * _V_PER_W

    tbuf = (t0, t1)
    obuf = (o0, o1)
    sin = (si0, si1)
    sout = (so0, so1)
    lanes = jax.lax.iota(jnp.int32, 16)
    hi = lanes + 16
    zeros = lanes * 0

    def transpose_block(tb, ob, width):
        @plsc.parallel_loop(0, width, 1, unroll=8)
        def _(vp):
            col = zeros + vp
            ob[pl.ds(vp * 32, 16)] = plsc.load_gather(tb, [lanes, col])
            ob[pl.ds(vp * 32 + 16, 16)] = plsc.load_gather(tb, [hi, col])

    def start_in(blk, par):
        pltpu.async_copy(
            tt_hbm.at[:, pl.ds(v_base + blk * _VW, _VW)],
            tbuf[par].at[:, pl.ds(0, _VW)],
            sin[par],
        )

    start_in(0, 0)
    start_in(1, 1)

    def step(j, carry):
        for par in range(2):
            blk = 2 * j + par
            pltpu.make_async_copy(
                tt_hbm.at[:, pl.ds(v_base, _VW)],
                tbuf[par].at[:, pl.ds(0, _VW)],
                sin[par],
            ).wait()

            @pl.when(j > 0)
            def _():
                pltpu.make_async_copy(
                    obuf[par], flat_hbm.at[pl.ds(v_base * 32, _VW * 32)],
                    sout[par],
                ).wait()

            transpose_block(tbuf[par], obuf[par], _VW)
            pltpu.async_copy(
                obuf[par],
                flat_hbm.at[pl.ds((v_base + blk * _VW) * 32, _VW * 32)],
                sout[par],
            )

            @pl.when(blk + 2 < _BLK_W - 1)
            def _():
                pltpu.async_copy(
                    tt_hbm.at[:, pl.ds(v_base + (blk + 2) * _VW, _VW)],
                    tbuf[par].at[:, pl.ds(0, _VW)],
                    sin[par],
                )

        return carry

    lax.fori_loop(0, (_BLK_W - 1) // 2, step, 0)
    for par in range(2):
        pltpu.make_async_copy(
            obuf[par], flat_hbm.at[pl.ds(v_base * 32, _VW * 32)], sout[par]
        ).wait()

    v0p = v_base + (_BLK_W - 1) * _VW
    pltpu.sync_copy(
        tt_hbm.at[:, pl.ds(v0p, _VW)], tbuf[0].at[:, pl.ds(0, _VW)]
    )
    transpose_block(tbuf[0], obuf[0], _VW)
    pltpu.sync_copy(obuf[0], flat_hbm.at[pl.ds(v0p * 32, _VW * 32)])

    for k in range(4):

        @pl.when(wid == k)
        def _():
            v0 = _TAIL_V0 + _TW * k
            pltpu.sync_copy(
                tt_hbm.at[:, pl.ds(v0, _TW)], tbuf[0].at[:, pl.ds(0, _TW)]
            )
            transpose_block(tbuf[0], obuf[0], _TW)
            pltpu.sync_copy(
                obuf[0].at[pl.ds(0, _TW * 32)],
                flat_hbm.at[pl.ds(v0 * 32, _TW * 32)],
            )


_NB = 256
_NQ = SEQ_LEN * (BATCH // _NB)
_NQW = _NQ // _NW


def _bc_body(t128_hbm, ids_hbm, out_hbm,
             r0, r1, ob0, ob1, ix, rx0, rx1, pb0, pb1, g0, g1, w0, w1):
    cid = lax.axis_index("c")
    sid = lax.axis_index("s")
    wid = sid * 2 + cid

    rows = (r0, r1)
    ob = (ob0, ob1)
    rx = (rx0, rx1)
    pb = (pb0, pb1)
    gsem = (g0, g1)
    wsem = (w0, w1)
    lanes = jax.lax.iota(jnp.int32, 16)

    def locate(i):
        q = wid + _NW * i
        s = q // (BATCH // _NB)
        b0 = (q % (BATCH // _NB)) * _NB
        return s, b0

    def prep(i, nb):
        s, b0 = locate(i)
        srow = s % 8
        pltpu.sync_copy(
            ids_hbm.at[pl.ds((s // 8) * 8, 8), pl.ds(b0, _NB)], ix
        )

        @plsc.parallel_loop(0, _NB // 16, 1, unroll=2)
        def _(g):
            v = ix[srow, pl.ds(g * 16, 16)]
            rx[nb][pl.ds(g * 16, 16)] = v >> 2
            pb[nb][pl.ds(g * 16, 16)] = (v & 3) * 32

        pltpu.async_copy(t128_hbm.at[rx[nb]], rows[nb], gsem[nb])

    def extract(b):
        @plsc.parallel_loop(0, _NB // 16, 1, unroll=1)
        def _(g):
            rowi = lanes + g * 16
            pc = pb[b][pl.ds(g * 16, 16)]
            for c in range(EMBEDDING_DIM):
                ob[b][c, pl.ds(g * 16, 16)] = plsc.load_gather(
                    rows[b], [rowi, pc + c]
                )

    def start_write(i, b):
        s, b0 = locate(i)
        pltpu.async_copy(ob[b], out_hbm.at[s, :, pl.ds(b0, _NB)], wsem[b])

    def drain(sem, dummy_b):
        pltpu.make_async_copy(
            ob[dummy_b], out_hbm.at[0, :, pl.ds(0, _NB)], sem
        ).wait()

    prep(0, 0)

    def step(j, carry):
        for par in range(2):
            i = 2 * j + par
            nb = (par + 1) % 2

            @pl.when(i + 1 < _NQW)
            def _():
                prep(i + 1, nb)

            pltpu.make_async_copy(
                t128_hbm.at[rx[par]], rows[par], gsem[par]
            ).wait()

            @pl.when(i >= 2)
            def _():
                drain(wsem[par], par)

            extract(par)
            start_write(i, par)
        return carry

    lax.fori_loop(0, _NQW // 2, step, 0)
    drain(wsem[0], 0)
    drain(wsem[1], 1)


def _gather_body(idx_hbm, table_hbm, out_hbm, idx_all, rows0, rows1,
                 g0, g1, w0, w1):
    cid = lax.axis_index("c")
    sid = lax.axis_index("s")
    wid = sid * 2 + cid
    w_base = wid * _PER_W

    pltpu.sync_copy(idx_hbm.at[wid], idx_all)

    rows = (rows0, rows1)
    gsem = (g0, g1)
    wsem = (w0, w1)
    gdesc = [None] * _NCHUNK
    wdesc = [None] * _NCHUNK
    gdesc[0] = pltpu.async_copy(table_hbm.at[idx_all.at[0]], rows[0], gsem[0])
    gdesc[1] = pltpu.async_copy(table_hbm.at[idx_all.at[1]], rows[1], gsem[1])
    for j in range(_NCHUNK):
        b = j % 2
        gdesc[j].wait()
        wdesc[j] = pltpu.async_copy(
            rows[b], out_hbm.at[pl.ds(w_base + j * _CHUNK, _CHUNK)], wsem[b]
        )
        if j + 2 < _NCHUNK:
            wdesc[j].wait()
            gdesc[j + 2] = pltpu.async_copy(
                table_hbm.at[idx_all.at[j + 2]], rows[b], gsem[b]
            )
    wdesc[_NCHUNK - 2].wait()
    wdesc[_NCHUNK - 1].wait()


@jax.jit
def kernel(token_ids, embedding):
    mesh = plsc.VectorSubcoreMesh(core_axis_name="c", subcore_axis_name="s")

    flat_table = pl.kernel(
        _transpose_body,
        out_type=jax.ShapeDtypeStruct((NUM_EMBEDDINGS * EMBEDDING_DIM,),
                                      jnp.float32),
        mesh=mesh,
        scratch_types=[
            pltpu.VMEM((EMBEDDING_DIM, _VW + 1), jnp.float32),
            pltpu.VMEM((EMBEDDING_DIM, _VW + 1), jnp.float32),
            pltpu.VMEM((_VW * EMBEDDING_DIM,), jnp.float32),
            pltpu.VMEM((_VW * EMBEDDING_DIM,), jnp.float32),
            pltpu.SemaphoreType.DMA,
            pltpu.SemaphoreType.DMA,
            pltpu.SemaphoreType.DMA,
            pltpu.SemaphoreType.DMA,
        ],
        compiler_params=pltpu.CompilerParams(
            use_tc_tiling_on_sc=True, needs_layout_passes=False,
            disable_bounds_checks=True
        ),
    )(embedding.T)

    tail_v0 = _TAIL_V0 + 4 * _TW
    tail_rows = jax.lax.slice(
        embedding, (tail_v0, 0), (NUM_EMBEDDINGS, EMBEDDING_DIM)
    ).reshape(-1)
    flat_table = jax.lax.dynamic_update_slice(
        flat_table, tail_rows, (tail_v0 * EMBEDDING_DIM,)
    )

    t128 = flat_table.reshape(
        NUM_EMBEDDINGS * EMBEDDING_DIM // 128, 128
    )
    ids_t = token_ids.T

    outp = pl.kernel(
        _bc_body,
        out_type=jax.ShapeDtypeStruct((SEQ_LEN, EMBEDDING_DIM, BATCH),
                                      jnp.float32),
        mesh=mesh,
        scratch_types=[
            pltpu.VMEM((_NB, 128), jnp.float32),
            pltpu.VMEM((_NB, 128), jnp.float32),
            pltpu.VMEM((EMBEDDING_DIM, _NB), jnp.float32),
            pltpu.VMEM((EMBEDDING_DIM, _NB), jnp.float32),
            pltpu.VMEM((8, _NB), jnp.int32),
            pltpu.VMEM((_NB,), jnp.int32),
            pltpu.VMEM((_NB,), jnp.int32),
            pltpu.VMEM((_NB,), jnp.int32),
            pltpu.VMEM((_NB,), jnp.int32),
            pltpu.SemaphoreType.DMA,
            pltpu.SemaphoreType.DMA,
            pltpu.SemaphoreType.DMA,
            pltpu.SemaphoreType.DMA,
        ],
        compiler_params=pltpu.CompilerParams(
            use_tc_tiling_on_sc=True, needs_layout_passes=False,
            disable_bounds_checks=True
        ),
    )(t128, ids_t)
    return outp.transpose(2, 0, 1)


# speedup vs baseline: 1.1480x; 1.0264x over previous
import jax
import jax.numpy as jnp
from jax import lax
from jax.experimental import pallas as pl
from jax.experimental.pallas import tpu as pltpu
from jax.experimental.pallas import tpu_sc as plsc

NUM_EMBEDDINGS = 1000000
EMBEDDING_DIM = 32
BATCH = 4096
SEQ_LEN = 200

_NW = 32
_B = BATCH * SEQ_LEN
_PER_W = _B // _NW
_CHUNK = 1600
_NCHUNK = _PER_W // _CHUNK

_VW = 512
_BLK_W = 61
_V_PER_W = _VW * _BLK_W
_TAIL_V0 = _NW * _V_PER_W
_TW = 128


def _transpose_body(tt_hbm, flat_hbm, t0, t1, o0, o1, si0, si1, so0, so1):
    cid = lax.axis_index("c")
    sid = lax.axis_index("s")
    wid = sid * 2 + cid
    v_base = wid * _V_PER_W

    tbuf = (t0, t1)
    obuf = (o0, o1)
    sin = (si0, si1)
    sout = (so0, so1)
    lanes = jax.lax.iota(jnp.int32, 16)
    hi = lanes + 16
    zeros = lanes * 0

    def transpose_block(tb, ob, width):
        @plsc.parallel_loop(0, width, 1, unroll=8)
        def _(vp):
            col = zeros + vp
            ob[pl.ds(vp * 32, 16)] = plsc.load_gather(tb, [lanes, col])
            ob[pl.ds(vp * 32 + 16, 16)] = plsc.load_gather(tb, [hi, col])

    def start_in(blk, par):
        pltpu.async_copy(
            tt_hbm.at[:, pl.ds(v_base + blk * _VW, _VW)],
            tbuf[par].at[:, pl.ds(0, _VW)],
            sin[par],
        )

    start_in(0, 0)
    start_in(1, 1)

    def step(j, carry):
        for par in range(2):
            blk = 2 * j + par
            pltpu.make_async_copy(
                tt_hbm.at[:, pl.ds(v_base, _VW)],
                tbuf[par].at[:, pl.ds(0, _VW)],
                sin[par],
            ).wait()

            @pl.when(j > 0)
            def _():
                pltpu.make_async_copy(
                    obuf[par], flat_hbm.at[pl.ds(v_base * 32, _VW * 32)],
                    sout[par],
                ).wait()

            transpose_block(tbuf[par], obuf[par], _VW)
            pltpu.async_copy(
                obuf[par],
                flat_hbm.at[pl.ds((v_base + blk * _VW) * 32, _VW * 32)],
                sout[par],
            )

            @pl.when(blk + 2 < _BLK_W - 1)
            def _():
                pltpu.async_copy(
                    tt_hbm.at[:, pl.ds(v_base + (blk + 2) * _VW, _VW)],
                    tbuf[par].at[:, pl.ds(0, _VW)],
                    sin[par],
                )

        return carry

    lax.fori_loop(0, (_BLK_W - 1) // 2, step, 0)
    for par in range(2):
        pltpu.make_async_copy(
            obuf[par], flat_hbm.at[pl.ds(v_base * 32, _VW * 32)], sout[par]
        ).wait()

    v0p = v_base + (_BLK_W - 1) * _VW
    pltpu.sync_copy(
        tt_hbm.at[:, pl.ds(v0p, _VW)], tbuf[0].at[:, pl.ds(0, _VW)]
    )
    transpose_block(tbuf[0], obuf[0], _VW)
    pltpu.sync_copy(obuf[0], flat_hbm.at[pl.ds(v0p * 32, _VW * 32)])

    for k in range(4):

        @pl.when(wid == k)
        def _():
            v0 = _TAIL_V0 + _TW * k
            pltpu.sync_copy(
                tt_hbm.at[:, pl.ds(v0, _TW)], tbuf[0].at[:, pl.ds(0, _TW)]
            )
            transpose_block(tbuf[0], obuf[0], _TW)
            pltpu.sync_copy(
                obuf[0].at[pl.ds(0, _TW * 32)],
                flat_hbm.at[pl.ds(v0 * 32, _TW * 32)],
            )


_NB = 256
_NQ = SEQ_LEN * (BATCH // _NB)
_NQW = _NQ // _NW


def _bc_body(t128_hbm, ids_hbm, out_hbm,
             r0, r1, ob0, ob1, ix, rx0, rx1, pb0, pb1, g0, g1, w0, w1):
    cid = lax.axis_index("c")
    sid = lax.axis_index("s")
    wid = sid * 2 + cid

    rows = (r0, r1)
    ob = (ob0, ob1)
    rx = (rx0, rx1)
    pb = (pb0, pb1)
    gsem = (g0, g1)
    wsem = (w0, w1)
    lanes = jax.lax.iota(jnp.int32, 16)

    def locate(i):
        q = wid + _NW * i
        s = q // (BATCH // _NB)
        b0 = (q % (BATCH // _NB)) * _NB
        return s, b0

    def prep(i, nb):
        s, b0 = locate(i)
        srow = s % 8
        pltpu.sync_copy(
            ids_hbm.at[pl.ds((s // 8) * 8, 8), pl.ds(b0, _NB)], ix
        )

        @plsc.parallel_loop(0, _NB // 16, 1, unroll=2)
        def _(g):
            v = ix[srow, pl.ds(g * 16, 16)]
            rx[nb][pl.ds(g * 16, 16)] = v >> 2
            pb[nb][pl.ds(g * 16, 16)] = (v & 3) * 32

        pltpu.async_copy(t128_hbm.at[rx[nb]], rows[nb], gsem[nb])

    def extract(b):
        @plsc.parallel_loop(0, _NB // 16, 1, unroll=2)
        def _(g):
            rowi = lanes + g * 16
            pc = pb[b][pl.ds(g * 16, 16)]
            for c in range(EMBEDDING_DIM):
                ob[b][c, pl.ds(g * 16, 16)] = plsc.load_gather(
                    rows[b], [rowi, pc + c]
                )

    def start_write(i, b):
        s, b0 = locate(i)
        pltpu.async_copy(ob[b], out_hbm.at[s, :, pl.ds(b0, _NB)], wsem[b])

    def drain(sem, dummy_b):
        pltpu.make_async_copy(
            ob[dummy_b], out_hbm.at[0, :, pl.ds(0, _NB)], sem
        ).wait()

    prep(0, 0)

    def step(j, carry):
        for par in range(2):
            i = 2 * j + par
            nb = (par + 1) % 2

            @pl.when(i + 1 < _NQW)
            def _():
                prep(i + 1, nb)

            pltpu.make_async_copy(
                t128_hbm.at[rx[par]], rows[par], gsem[par]
            ).wait()

            @pl.when(i >= 2)
            def _():
                drain(wsem[par], par)

            extract(par)
            start_write(i, par)
        return carry

    lax.fori_loop(0, _NQW // 2, step, 0)
    drain(wsem[0], 0)
    drain(wsem[1], 1)


def _gather_body(idx_hbm, table_hbm, out_hbm, idx_all, rows0, rows1,
                 g0, g1, w0, w1):
    cid = lax.axis_index("c")
    sid = lax.axis_index("s")
    wid = sid * 2 + cid
    w_base = wid * _PER_W

    pltpu.sync_copy(idx_hbm.at[wid], idx_all)

    rows = (rows0, rows1)
    gsem = (g0, g1)
    wsem = (w0, w1)
    gdesc = [None] * _NCHUNK
    wdesc = [None] * _NCHUNK
    gdesc[0] = pltpu.async_copy(table_hbm.at[idx_all.at[0]], rows[0], gsem[0])
    gdesc[1] = pltpu.async_copy(table_hbm.at[idx_all.at[1]], rows[1], gsem[1])
    for j in range(_NCHUNK):
        b = j % 2
        gdesc[j].wait()
        wdesc[j] = pltpu.async_copy(
            rows[b], out_hbm.at[pl.ds(w_base + j * _CHUNK, _CHUNK)], wsem[b]
        )
        if j + 2 < _NCHUNK:
            wdesc[j].wait()
            gdesc[j + 2] = pltpu.async_copy(
                table_hbm.at[idx_all.at[j + 2]], rows[b], gsem[b]
            )
    wdesc[_NCHUNK - 2].wait()
    wdesc[_NCHUNK - 1].wait()


@jax.jit
def kernel(token_ids, embedding):
    mesh = plsc.VectorSubcoreMesh(core_axis_name="c", subcore_axis_name="s")

    flat_table = pl.kernel(
        _transpose_body,
        out_type=jax.ShapeDtypeStruct((NUM_EMBEDDINGS * EMBEDDING_DIM,),
                                      jnp.float32),
        mesh=mesh,
        scratch_types=[
            pltpu.VMEM((EMBEDDING_DIM, _VW + 1), jnp.float32),
            pltpu.VMEM((EMBEDDING_DIM, _VW + 1), jnp.float32),
            pltpu.VMEM((_VW * EMBEDDING_DIM,), jnp.float32),
            pltpu.VMEM((_VW * EMBEDDING_DIM,), jnp.float32),
            pltpu.SemaphoreType.DMA,
            pltpu.SemaphoreType.DMA,
            pltpu.SemaphoreType.DMA,
            pltpu.SemaphoreType.DMA,
        ],
        compiler_params=pltpu.CompilerParams(
            use_tc_tiling_on_sc=True, needs_layout_passes=False,
            disable_bounds_checks=True
        ),
    )(embedding.T)

    tail_v0 = _TAIL_V0 + 4 * _TW
    tail_rows = jax.lax.slice(
        embedding, (tail_v0, 0), (NUM_EMBEDDINGS, EMBEDDING_DIM)
    ).reshape(-1)
    flat_table = jax.lax.dynamic_update_slice(
        flat_table, tail_rows, (tail_v0 * EMBEDDING_DIM,)
    )

    t128 = flat_table.reshape(
        NUM_EMBEDDINGS * EMBEDDING_DIM // 128, 128
    )
    ids_t = token_ids.T

    outp = pl.kernel(
        _bc_body,
        out_type=jax.ShapeDtypeStruct((SEQ_LEN, EMBEDDING_DIM, BATCH),
                                      jnp.float32),
        mesh=mesh,
        scratch_types=[
            pltpu.VMEM((_NB, 128), jnp.float32),
            pltpu.VMEM((_NB, 128), jnp.float32),
            pltpu.VMEM((EMBEDDING_DIM, _NB), jnp.float32),
            pltpu.VMEM((EMBEDDING_DIM, _NB), jnp.float32),
            pltpu.VMEM((8, _NB), jnp.int32),
            pltpu.VMEM((_NB,), jnp.int32),
            pltpu.VMEM((_NB,), jnp.int32),
            pltpu.VMEM((_NB,), jnp.int32),
            pltpu.VMEM((_NB,), jnp.int32),
            pltpu.SemaphoreType.DMA,
            pltpu.SemaphoreType.DMA,
            pltpu.SemaphoreType.DMA,
            pltpu.SemaphoreType.DMA,
        ],
        compiler_params=pltpu.CompilerParams(
            use_tc_tiling_on_sc=True, needs_layout_passes=False,
            disable_bounds_checks=True
        ),
    )(t128, ids_t)
    return outp.transpose(2, 0, 1)


# speedup vs baseline: 1.1484x; 1.0004x over previous
import jax
import jax.numpy as jnp
from jax import lax
from jax.experimental import pallas as pl
from jax.experimental.pallas import tpu as pltpu
from jax.experimental.pallas import tpu_sc as plsc

NUM_EMBEDDINGS = 1000000
EMBEDDING_DIM = 32
BATCH = 4096
SEQ_LEN = 200

_NW = 32

_VW = 512
_BLK_W = 61
_V_PER_W = _VW * _BLK_W
_TAIL_V0 = _NW * _V_PER_W
_TW = 128


def _transpose_body(tt_hbm, flat_hbm, t0, t1, o0, o1, si0, si1, so0, so1):
    cid = lax.axis_index("c")
    sid = lax.axis_index("s")
    wid = sid * 2 + cid
    v_base = wid * _V_PER_W

    tbuf = (t0, t1)
    obuf = (o0, o1)
    sin = (si0, si1)
    sout = (so0, so1)
    lanes = jax.lax.iota(jnp.int32, 16)
    hi = lanes + 16
    zeros = lanes * 0

    def transpose_block(tb, ob, width):
        @plsc.parallel_loop(0, width, 1, unroll=8)
        def _(vp):
            col = zeros + vp
            ob[pl.ds(vp * 32, 16)] = plsc.load_gather(tb, [lanes, col])
            ob[pl.ds(vp * 32 + 16, 16)] = plsc.load_gather(tb, [hi, col])

    def start_in(blk, par):
        pltpu.async_copy(
            tt_hbm.at[:, pl.ds(v_base + blk * _VW, _VW)],
            tbuf[par].at[:, pl.ds(0, _VW)],
            sin[par],
        )

    start_in(0, 0)
    start_in(1, 1)

    def step(j, carry):
        for par in range(2):
            blk = 2 * j + par
            pltpu.make_async_copy(
                tt_hbm.at[:, pl.ds(v_base, _VW)],
                tbuf[par].at[:, pl.ds(0, _VW)],
                sin[par],
            ).wait()

            @pl.when(j > 0)
            def _():
                pltpu.make_async_copy(
                    obuf[par], flat_hbm.at[pl.ds(v_base * 32, _VW * 32)],
                    sout[par],
                ).wait()

            transpose_block(tbuf[par], obuf[par], _VW)
            pltpu.async_copy(
                obuf[par],
                flat_hbm.at[pl.ds((v_base + blk * _VW) * 32, _VW * 32)],
                sout[par],
            )

            @pl.when(blk + 2 < _BLK_W - 1)
            def _():
                pltpu.async_copy(
                    tt_hbm.at[:, pl.ds(v_base + (blk + 2) * _VW, _VW)],
                    tbuf[par].at[:, pl.ds(0, _VW)],
                    sin[par],
                )

        return carry

    lax.fori_loop(0, (_BLK_W - 1) // 2, step, 0)
    for par in range(2):
        pltpu.make_async_copy(
            obuf[par], flat_hbm.at[pl.ds(v_base * 32, _VW * 32)], sout[par]
        ).wait()

    v0p = v_base + (_BLK_W - 1) * _VW
    pltpu.sync_copy(
        tt_hbm.at[:, pl.ds(v0p, _VW)], tbuf[0].at[:, pl.ds(0, _VW)]
    )
    transpose_block(tbuf[0], obuf[0], _VW)
    pltpu.sync_copy(obuf[0], flat_hbm.at[pl.ds(v0p * 32, _VW * 32)])

    for k in range(4):

        @pl.when(wid == k)
        def _():
            v0 = _TAIL_V0 + _TW * k
            pltpu.sync_copy(
                tt_hbm.at[:, pl.ds(v0, _TW)], tbuf[0].at[:, pl.ds(0, _TW)]
            )
            transpose_block(tbuf[0], obuf[0], _TW)
            pltpu.sync_copy(
                obuf[0].at[pl.ds(0, _TW * 32)],
                flat_hbm.at[pl.ds(v0 * 32, _TW * 32)],
            )


_NB = 256
_NQ = SEQ_LEN * (BATCH // _NB)
_NQW = _NQ // _NW


def _bc_body(t128_hbm, ids_hbm, out_hbm,
             r0, r1, ob0, ob1, ix, rx0, rx1, pb0, pb1, g0, g1, w0, w1):
    cid = lax.axis_index("c")
    sid = lax.axis_index("s")
    wid = sid * 2 + cid

    rows = (r0, r1)
    ob = (ob0, ob1)
    rx = (rx0, rx1)
    pb = (pb0, pb1)
    gsem = (g0, g1)
    wsem = (w0, w1)
    lanes = jax.lax.iota(jnp.int32, 16)

    def locate(i):
        q = wid + _NW * i
        s = q // (BATCH // _NB)
        b0 = (q % (BATCH // _NB)) * _NB
        return s, b0

    def prep(i, nb):
        s, b0 = locate(i)
        srow = s % 8
        pltpu.sync_copy(
            ids_hbm.at[pl.ds((s // 8) * 8, 8), pl.ds(b0, _NB)], ix
        )

        @plsc.parallel_loop(0, _NB // 16, 1, unroll=2)
        def _(g):
            v = ix[srow, pl.ds(g * 16, 16)]
            rx[nb][pl.ds(g * 16, 16)] = v >> 2
            pb[nb][pl.ds(g * 16, 16)] = (v & 3) * 32

        pltpu.async_copy(t128_hbm.at[rx[nb]], rows[nb], gsem[nb])

    def extract(b):
        @plsc.parallel_loop(0, _NB // 16, 1, unroll=2)
        def _(g):
            rowi = lanes + g * 16
            pc = pb[b][pl.ds(g * 16, 16)]
            for c in range(EMBEDDING_DIM):
                ob[b][c, pl.ds(g * 16, 16)] = plsc.load_gather(
                    rows[b], [rowi, pc + c]
                )

    def start_write(i, b):
        s, b0 = locate(i)
        pltpu.async_copy(ob[b], out_hbm.at[s, :, pl.ds(b0, _NB)], wsem[b])

    def drain(sem, dummy_b):
        pltpu.make_async_copy(
            ob[dummy_b], out_hbm.at[0, :, pl.ds(0, _NB)], sem
        ).wait()

    prep(0, 0)

    def step(j, carry):
        for par in range(2):
            i = 2 * j + par
            nb = (par + 1) % 2

            @pl.when(i + 1 < _NQW)
            def _():
                prep(i + 1, nb)

            pltpu.make_async_copy(
                t128_hbm.at[rx[par]], rows[par], gsem[par]
            ).wait()

            @pl.when(i >= 2)
            def _():
                drain(wsem[par], par)

            extract(par)
            start_write(i, par)
        return carry

    lax.fori_loop(0, _NQW // 2, step, 0)
    drain(wsem[0], 0)
    drain(wsem[1], 1)


@jax.jit
def kernel(token_ids, embedding):
    mesh = plsc.VectorSubcoreMesh(core_axis_name="c", subcore_axis_name="s")

    flat_table = pl.kernel(
        _transpose_body,
        out_type=jax.ShapeDtypeStruct((NUM_EMBEDDINGS * EMBEDDING_DIM,),
                                      jnp.float32),
        mesh=mesh,
        scratch_types=[
            pltpu.VMEM((EMBEDDING_DIM, _VW + 1), jnp.float32),
            pltpu.VMEM((EMBEDDING_DIM, _VW + 1), jnp.float32),
            pltpu.VMEM((_VW * EMBEDDING_DIM,), jnp.float32),
            pltpu.VMEM((_VW * EMBEDDING_DIM,), jnp.float32),
            pltpu.SemaphoreType.DMA,
            pltpu.SemaphoreType.DMA,
            pltpu.SemaphoreType.DMA,
            pltpu.SemaphoreType.DMA,
        ],
        compiler_params=pltpu.CompilerParams(
            use_tc_tiling_on_sc=True, needs_layout_passes=False,
            disable_bounds_checks=True
        ),
    )(embedding.T)

    tail_v0 = _TAIL_V0 + 4 * _TW
    tail_rows = jax.lax.slice(
        embedding, (tail_v0, 0), (NUM_EMBEDDINGS, EMBEDDING_DIM)
    ).reshape(-1)
    flat_table = jax.lax.dynamic_update_slice(
        flat_table, tail_rows, (tail_v0 * EMBEDDING_DIM,)
    )

    t128 = flat_table.reshape(
        NUM_EMBEDDINGS * EMBEDDING_DIM // 128, 128
    )
    ids_t = token_ids.T

    outp = pl.kernel(
        _bc_body,
        out_type=jax.ShapeDtypeStruct((SEQ_LEN, EMBEDDING_DIM, BATCH),
                                      jnp.float32),
        mesh=mesh,
        scratch_types=[
            pltpu.VMEM((_NB, 128), jnp.float32),
            pltpu.VMEM((_NB, 128), jnp.float32),
            pltpu.VMEM((EMBEDDING_DIM, _NB), jnp.float32),
            pltpu.VMEM((EMBEDDING_DIM, _NB), jnp.float32),
            pltpu.VMEM((8, _NB), jnp.int32),
            pltpu.VMEM((_NB,), jnp.int32),
            pltpu.VMEM((_NB,), jnp.int32),
            pltpu.VMEM((_NB,), jnp.int32),
            pltpu.VMEM((_NB,), jnp.int32),
            pltpu.SemaphoreType.DMA,
            pltpu.SemaphoreType.DMA,
            pltpu.SemaphoreType.DMA,
            pltpu.SemaphoreType.DMA,
        ],
        compiler_params=pltpu.CompilerParams(
            use_tc_tiling_on_sc=True, needs_layout_passes=False,
            disable_bounds_checks=True
        ),
    )(t128, ids_t)
    return outp.transpose(2, 0, 1)
